# Initial kernel scaffold; baseline (speedup 1.0000x reference)
import jax
import jax.numpy as jnp
from jax.experimental import pallas as pl


def kernel(x, edge_index, W1, b1, W2, b2, lin1_W, lin1_b, linf_W, linf_b):
    raise NotImplementedError("write your pallas kernel here")


# speedup vs baseline: 7.9223x; 7.9223x over previous
import functools

import jax
import jax.numpy as jnp
from jax import lax
from jax.experimental import pallas as pl
from jax.experimental.pallas import tpu as pltpu
from jax.experimental.pallas import tpu_sc as plsc

N = 10000
NP = 10240
D = 128
E = 640000
C = 2
K = 128
NC, NS = 2, 16
NW = NC * NS
CHUNKS = -(-E // (NW * K))
EP = CHUNKS * NW * K
EPW = CHUNKS * K
RPT = NP // NS

_mesh = plsc.VectorSubcoreMesh(core_axis_name="c", subcore_axis_name="s",
                               num_cores=NC, num_subcores=NS)
_f32 = jnp.float32


def _fill2d(ref, rows, cols, val):
    def row(i, _):
        def col(j, _):
            ref[i, pl.ds(j * 16, 16)] = jnp.full((16,), val, _f32)
            return 0
        return lax.fori_loop(0, cols // 16, col, 0)
    lax.fori_loop(0, rows, row, 0)


def _deg_body(dst_hbm, out_hbm, idx_v, ones_v, zb_v, acc_sh):
    c = lax.axis_index("c")
    s = lax.axis_index("s")
    w = s * NC + c
    _fill2d(ones_v, K, D, 1.0)
    _fill2d(zb_v, 64, D, 0.0)

    def z(i, _):
        pltpu.sync_copy(zb_v, acc_sh.at[pl.ds(s * RPT + i * 64, 64)])
        return 0
    lax.fori_loop(0, RPT // 64, z, 0)
    plsc.subcore_barrier()

    def step(k, _):
        base = w * EPW + k * K
        pltpu.sync_copy(dst_hbm.at[pl.ds(base, K)], idx_v)
        pltpu.sync_copy(ones_v, acc_sh.at[idx_v], add=True)
        return 0
    lax.fori_loop(0, CHUNKS, step, 0)
    plsc.subcore_barrier()
    pltpu.sync_copy(acc_sh.at[pl.ds(s * RPT, RPT)],
                    out_hbm.at[c, pl.ds(s * RPT, RPT)])


_deg_call = functools.partial(
    pl.kernel,
    out_type=jax.ShapeDtypeStruct((NC, NP, D), _f32),
    mesh=_mesh,
    scratch_types=[
        pltpu.VMEM((K,), jnp.int32),
        pltpu.VMEM((K, D), _f32),
        pltpu.VMEM((64, D), _f32),
        pltpu.VMEM_SHARED((NP, D), _f32),
    ],
)(_deg_body)


def _agg_body(g_hbm, src_hbm, dst_hbm, out_hbm,
              sidx_v, didx_v, rows_v, zb_v, acc_sh, sem):
    c = lax.axis_index("c")
    s = lax.axis_index("s")
    w = s * NC + c
    _fill2d(zb_v, 64, D, 0.0)

    def z(i, _):
        pltpu.sync_copy(zb_v, acc_sh.at[pl.ds(s * RPT + i * 64, 64)])
        return 0
    lax.fori_loop(0, RPT // 64, z, 0)
    plsc.subcore_barrier()

    def step(k, _):
        base = w * EPW + k * K
        pltpu.sync_copy(src_hbm.at[pl.ds(base, K)], sidx_v)
        pltpu.sync_copy(dst_hbm.at[pl.ds(base, K)], didx_v)
        pltpu.async_copy(g_hbm.at[sidx_v], rows_v, sem).wait()
        pltpu.sync_copy(rows_v, acc_sh.at[didx_v], add=True)
        return 0
    lax.fori_loop(0, CHUNKS, step, 0)
    plsc.subcore_barrier()
    pltpu.sync_copy(acc_sh.at[pl.ds(s * RPT, RPT)],
                    out_hbm.at[c, pl.ds(s * RPT, RPT)])


_agg_call = functools.partial(
    pl.kernel,
    out_type=jax.ShapeDtypeStruct((NC, NP, D), _f32),
    mesh=_mesh,
    scratch_types=[
        pltpu.VMEM((K,), jnp.int32),
        pltpu.VMEM((K,), jnp.int32),
        pltpu.VMEM((K, D), _f32),
        pltpu.VMEM((64, D), _f32),
        pltpu.VMEM_SHARED((NP, D), _f32),
        pltpu.SemaphoreType.DMA,
    ],
)(_agg_body)


def _head_body(a_hbm, b_hbm, src_hbm, dst_hbm, out_hbm,
               sidx_v, didx_v, ra_v, rb_v, iota_v, stage_sh, sem_a, sem_b):
    c = lax.axis_index("c")
    s = lax.axis_index("s")
    w = s * NC + c

    def f(j, _):
        iota_v[pl.ds(j * 16, 16)] = lax.iota(jnp.int32, 16) + (s * K + j * 16)
        return 0
    lax.fori_loop(0, K // 16, f, 0)

    def step(k, _):
        base = w * EPW + k * K
        pltpu.sync_copy(src_hbm.at[pl.ds(base, K)], sidx_v)
        pltpu.sync_copy(dst_hbm.at[pl.ds(base, K)], didx_v)
        cp_a = pltpu.async_copy(a_hbm.at[sidx_v], ra_v, sem_a)
        cp_b = pltpu.async_copy(b_hbm.at[didx_v], rb_v, sem_b)
        cp_a.wait()
        cp_b.wait()
        pltpu.sync_copy(ra_v, stage_sh.at[pl.ds(s * K, K)])
        pltpu.sync_copy(rb_v, stage_sh.at[iota_v], add=True)
        pltpu.sync_copy(stage_sh.at[pl.ds(s * K, K)], out_hbm.at[pl.ds(base, K)])
        return 0
    lax.fori_loop(0, CHUNKS, step, 0)


_head_call = functools.partial(
    pl.kernel,
    out_type=jax.ShapeDtypeStruct((EP, D), _f32),
    mesh=_mesh,
    scratch_types=[
        pltpu.VMEM((K,), jnp.int32),
        pltpu.VMEM((K,), jnp.int32),
        pltpu.VMEM((K, D), _f32),
        pltpu.VMEM((K, D), _f32),
        pltpu.VMEM((K,), jnp.int32),
        pltpu.VMEM_SHARED((NS * K, D), _f32),
        pltpu.SemaphoreType.DMA,
        pltpu.SemaphoreType.DMA,
    ],
)(_head_body)


def _dis(degp_ref):
    deg = degp_ref[0, :, 0:1] + degp_ref[1, :, 0:1] + 1.0
    return lax.rsqrt(deg)


def _s1_body(x_ref, w1_ref, degp_ref, g1_ref):
    dis = _dis(degp_ref)
    hw = jnp.dot(x_ref[...], w1_ref[...], preferred_element_type=_f32)
    g1_ref[...] = hw * dis


def _s2_body(agg_ref, x_ref, w1_ref, b1_ref, w2_ref, degp_ref, g2_ref, hw2_ref):
    dis = _dis(degp_ref)
    hw1 = jnp.dot(x_ref[...], w1_ref[...], preferred_element_type=_f32)
    h1 = jnp.maximum(dis * (agg_ref[0] + agg_ref[1]) + dis * dis * hw1
                     + b1_ref[...], 0.0)
    hw2 = jnp.dot(h1, w2_ref[...], preferred_element_type=_f32)
    hw2_ref[...] = hw2
    g2_ref[...] = hw2 * dis


def _s3_body(agg_ref, hw2_ref, b2_ref, w1a_ref, w1b_ref, l1b_ref, degp_ref,
             a_ref, b_ref):
    dis = _dis(degp_ref)
    h2 = jnp.maximum(dis * (agg_ref[0] + agg_ref[1]) + dis * dis * hw2_ref[...]
                     + b2_ref[...], 0.0)
    a_ref[...] = jnp.dot(h2, w1a_ref[...], preferred_element_type=_f32) + l1b_ref[...]
    b_ref[...] = jnp.dot(h2, w1b_ref[...], preferred_element_type=_f32)


BE = 4096


def _s4_body(s_ref, wf_ref, bf_ref, o_ref):
    t = jnp.maximum(s_ref[...], 0.0)
    z = jnp.dot(t, wf_ref[...], preferred_element_type=_f32) + bf_ref[...]
    m = jnp.max(z, axis=1, keepdims=True)
    o_ref[...] = z - m - jnp.log(jnp.sum(jnp.exp(z - m), axis=1, keepdims=True))


def kernel(x, edge_index, W1, b1, W2, b2, lin1_W, lin1_b, linf_W, linf_b):
    src = edge_index[0]
    dst = edge_index[1]
    pad = jnp.full((EP - E,), NP - 1, jnp.int32)
    src_p = jnp.concatenate([src, pad])
    dst_p = jnp.concatenate([dst, pad])
    x_p = jnp.concatenate([x, jnp.zeros((NP - N, x.shape[1]), _f32)])

    degp = _deg_call(dst_p)

    g1 = pl.pallas_call(
        _s1_body,
        out_shape=jax.ShapeDtypeStruct((NP, D), _f32),
    )(x_p, W1, degp)

    agg1 = _agg_call(g1, src_p, dst_p)

    g2, hw2 = pl.pallas_call(
        _s2_body,
        out_shape=[jax.ShapeDtypeStruct((NP, D), _f32),
                   jax.ShapeDtypeStruct((NP, D), _f32)],
    )(agg1, x_p, W1, b1.reshape(1, D), W2, degp)

    agg2 = _agg_call(g2, src_p, dst_p)

    A, B = pl.pallas_call(
        _s3_body,
        out_shape=[jax.ShapeDtypeStruct((NP, D), _f32),
                   jax.ShapeDtypeStruct((NP, D), _f32)],
    )(agg2, hw2, b2.reshape(1, D), lin1_W[:D], lin1_W[D:], lin1_b.reshape(1, D),
      degp)

    s = _head_call(A, B, src_p, dst_p)

    outp = pl.pallas_call(
        _s4_body,
        grid=(EP // BE,),
        in_specs=[
            pl.BlockSpec((BE, D), lambda i: (i, 0)),
            pl.BlockSpec((D, C), lambda i: (0, 0)),
            pl.BlockSpec((1, C), lambda i: (0, 0)),
        ],
        out_specs=pl.BlockSpec((BE, C), lambda i: (i, 0)),
        out_shape=jax.ShapeDtypeStruct((EP, C), _f32),
    )(s, linf_W, linf_b.reshape(1, C))

    return lax.slice(outp, (0, 0), (E, C))
